# baseline (speedup 1.0000x reference)
import functools

import jax
import jax.numpy as jnp
from jax import lax
from jax.experimental import pallas as pl
from jax.experimental.pallas import tpu as pltpu
from jax.experimental.pallas import tpu_sc as plsc

RES = 256
NBINS = RES * RES
NUM_ROOMS = 16
N_AGENTS = 200000

NUM_CORES = 2
NUM_SUBCORES = 16
NW = NUM_CORES * NUM_SUBCORES
VALID = N_AGENTS // NW
ROWS = 49
FULL_ROWS = ROWS - 1
TAIL_VALID = VALID - FULL_ROWS * 128
LOAD_LEN = VALID + 6
CHUNK = ROWS * 128 + 16
ZCH = NBINS // NUM_SUBCORES


def _sc_histogram(pos_flat):
    mesh = plsc.VectorSubcoreMesh(core_axis_name="c", subcore_axis_name="s")

    @functools.partial(
        pl.kernel,
        out_type=jax.ShapeDtypeStruct((NUM_CORES, NBINS), jnp.float32),
        mesh=mesh,
        scratch_types=[
            pltpu.VMEM((2 * LOAD_LEN + 64,), jnp.float32),
            pltpu.VMEM((ROWS, 128), jnp.int32),
            pltpu.VMEM((128,), jnp.float32),
            pltpu.VMEM((128,), jnp.float32),
            pltpu.VMEM((ZCH,), jnp.float32),
            pltpu.VMEM_SHARED((NBINS,), jnp.float32),
            pltpu.SemaphoreType.DMA,
            pltpu.SemaphoreType.DMA,
        ],
    )
    def hist_kernel(pos_hbm, out_hbm, xy_v, idx_v, ones_v, tail_v,
                    zero_v, hist_sh, sem, sem_sc):
        cid = lax.axis_index("c")
        sid = lax.axis_index("s")
        wid = sid * NUM_CORES + cid
        base = wid * VALID
        sh = lax.rem(base, 8)
        base_al = pl.multiple_of(base - sh, 8)

        cp_xy = pltpu.make_async_copy(
            pos_hbm.at[pl.ds(2 * base_al, 2 * LOAD_LEN)],
            xy_v.at[pl.ds(0, 2 * LOAD_LEN)], sem)
        cp_xy.start()

        @pl.loop(0, 128, step=16)
        def _(i):
            ones_v[pl.ds(i, 16)] = jnp.full((16,), 1.0, jnp.float32)
            lane = lax.iota(jnp.int32, 16) + i
            tail_v[pl.ds(i, 16)] = jnp.where(lane < TAIL_VALID, 1.0, 0.0)

        @pl.loop(0, ZCH, step=16)
        def _(i):
            zero_v[pl.ds(i, 16)] = jnp.zeros((16,), jnp.float32)

        pltpu.sync_copy(zero_v, hist_sh.at[pl.ds(sid * ZCH, ZCH)])
        plsc.subcore_barrier()

        cp_xy.wait()

        lane = lax.iota(jnp.int32, 16)
        wvec = jnp.where(lax.rem(lane, 2) == 0, RES, 1)
        shift1 = jnp.minimum(lane + 1, 15)
        evens = lax.rem(lane * 2, 16)
        lo8 = lane < 8

        def pair_bins(v):
            t = (v * 256.0).astype(jnp.int32)
            t = jnp.minimum(jnp.maximum(t, 0), RES - 1)
            sc = t * wvec
            return sc + sc.at[shift1].get(mode="promise_in_bounds")

        def compute_row(r):
            for c in range(8):
                foff = 2 * (sh + r * 128 + c * 16)
                sa = pair_bins(xy_v[pl.ds(foff, 16)])
                sb = pair_bins(xy_v[pl.ds(foff + 16, 16)])
                ca = sa.at[evens].get(mode="promise_in_bounds")
                cb = sb.at[evens].get(mode="promise_in_bounds")
                idx_v[r, pl.ds(c * 16, 16)] = jnp.where(lo8, ca, cb)

        @pl.loop(0, FULL_ROWS)
        def _(r):
            compute_row(r)
            pltpu.async_copy(ones_v, hist_sh.at[idx_v.at[r]], sem_sc,
                             add=True)

        compute_row(FULL_ROWS)
        pltpu.async_copy(tail_v, hist_sh.at[idx_v.at[FULL_ROWS]], sem_sc,
                         add=True)

        @pl.loop(0, ROWS)
        def _(r):
            pltpu.make_async_copy(ones_v, hist_sh.at[idx_v.at[r]],
                                  sem_sc).wait()

        plsc.subcore_barrier()

        pltpu.sync_copy(hist_sh.at[pl.ds(sid * ZCH, ZCH)],
                        out_hbm.at[cid, pl.ds(sid * ZCH, ZCH)])

    return hist_kernel(pos_flat)


def _layout_body(rp_ref, wall_ref, out_ref):
    r = pl.program_id(0)
    cx = rp_ref[r, 0]
    cy = rp_ref[r, 1]
    sx = rp_ref[r, 2]
    sy = rp_ref[r, 3]
    xi = lax.broadcasted_iota(jnp.int32, (RES, RES), 0).astype(jnp.float32) * (
        1.0 / (RES - 1))
    yj = lax.broadcasted_iota(jnp.int32, (RES, RES), 1).astype(jnp.float32) * (
        1.0 / (RES - 1))
    dx = xi - cx
    dy = yj - cy
    e = jnp.exp(-(dx * dx / (2.0 * sx * sx) + dy * dy / (2.0 * sy * sy)))
    out_ref[0] = e * (1.0 - wall_ref[...])


def _tc_layout(room_params, wall_density):
    return pl.pallas_call(
        _layout_body,
        grid=(NUM_ROOMS,),
        in_specs=[
            pl.BlockSpec(memory_space=pltpu.SMEM),
            pl.BlockSpec((RES, RES), lambda r: (0, 0)),
        ],
        out_specs=pl.BlockSpec((1, RES, RES), lambda r: (r, 0, 0)),
        out_shape=jax.ShapeDtypeStruct((NUM_ROOMS, RES, RES), jnp.float32),
    )(room_params, wall_density)


def _flow_body(p_ref, out_ref):
    h = p_ref[0] + p_ref[1]
    m = jnp.max(h)
    out_ref[...] = h / (m + 1e-6)


def _tc_flow(partial):
    return pl.pallas_call(
        _flow_body,
        out_shape=jax.ShapeDtypeStruct((RES, RES), jnp.float32),
    )(partial)


def kernel(agent_positions, room_params, wall_density):
    partial = _sc_histogram(agent_positions.reshape(-1))
    dynamic_layout = _tc_layout(room_params, wall_density)
    flow_field = _tc_flow(partial.reshape(NUM_CORES, RES, RES))
    return dynamic_layout, flow_field


# speedup vs baseline: 5.4245x; 5.4245x over previous
import functools

import jax
import jax.numpy as jnp
from jax import lax
from jax.experimental import pallas as pl
from jax.experimental.pallas import tpu as pltpu
from jax.experimental.pallas import tpu_sc as plsc

RES = 256
NBINS = RES * RES
NUM_ROOMS = 16
N_AGENTS = 200000

NUM_CORES = 2
NUM_SUBCORES = 16
NW = NUM_CORES * NUM_SUBCORES
ROWS = 49
CHUNK = ROWS * 128
PAD_N = NW * CHUNK
PAD_COUNT = PAD_N - N_AGENTS
ZCH = NBINS // NUM_SUBCORES


def _sc_histogram(pos_t):
    mesh = plsc.VectorSubcoreMesh(core_axis_name="c", subcore_axis_name="s")

    @functools.partial(
        pl.kernel,
        out_type=jax.ShapeDtypeStruct((NUM_CORES, NBINS), jnp.float32),
        mesh=mesh,
        scratch_types=[
            pltpu.VMEM((CHUNK,), jnp.float32),
            pltpu.VMEM((CHUNK,), jnp.float32),
            pltpu.VMEM((ROWS, 128), jnp.int32),
            pltpu.VMEM((128,), jnp.float32),
            pltpu.VMEM((ZCH,), jnp.float32),
            pltpu.VMEM_SHARED((NBINS,), jnp.float32),
            pltpu.SemaphoreType.DMA,
            pltpu.SemaphoreType.DMA,
        ],
    )
    def hist_kernel(pos_hbm, out_hbm, x_v, y_v, idx_v, ones_v, zero_v,
                    hist_sh, sem, sem_sc):
        cid = lax.axis_index("c")
        sid = lax.axis_index("s")
        wid = sid * NUM_CORES + cid
        base = wid * CHUNK

        cp_x = pltpu.make_async_copy(pos_hbm.at[0, pl.ds(base, CHUNK)], x_v,
                                     sem)
        cp_y = pltpu.make_async_copy(pos_hbm.at[1, pl.ds(base, CHUNK)], y_v,
                                     sem)
        cp_x.start()
        cp_y.start()

        @pl.loop(0, 128, step=16)
        def _(i):
            ones_v[pl.ds(i, 16)] = jnp.full((16,), 1.0, jnp.float32)

        @pl.loop(0, ZCH, step=16)
        def _(i):
            zero_v[pl.ds(i, 16)] = jnp.zeros((16,), jnp.float32)

        pltpu.sync_copy(zero_v, hist_sh.at[pl.ds(sid * ZCH, ZCH)])
        plsc.subcore_barrier()

        cp_x.wait()
        cp_y.wait()

        @pl.loop(0, ROWS)
        def _(r):
            for c in range(8):
                off = r * 128 + c * 16
                x = x_v[pl.ds(off, 16)]
                y = y_v[pl.ds(off, 16)]
                ix = (x * 256.0).astype(jnp.int32)
                iy = (y * 256.0).astype(jnp.int32)
                ix = jnp.minimum(jnp.maximum(ix, 0), RES - 1)
                iy = jnp.minimum(jnp.maximum(iy, 0), RES - 1)
                idx_v[r, pl.ds(c * 16, 16)] = ix * RES + iy
            pltpu.async_copy(ones_v, hist_sh.at[idx_v.at[r]], sem_sc,
                             add=True)

        @pl.loop(0, ROWS)
        def _(r):
            pltpu.make_async_copy(ones_v, hist_sh.at[idx_v.at[r]],
                                  sem_sc).wait()

        plsc.subcore_barrier()

        pltpu.sync_copy(hist_sh.at[pl.ds(sid * ZCH, ZCH)],
                        out_hbm.at[cid, pl.ds(sid * ZCH, ZCH)])

    return hist_kernel(pos_t)


def _layout_body(rp_ref, wall_ref, out_ref):
    xi = lax.broadcasted_iota(jnp.int32, (RES, RES), 0).astype(jnp.float32) * (
        1.0 / (RES - 1))
    yj = lax.broadcasted_iota(jnp.int32, (RES, RES), 1).astype(jnp.float32) * (
        1.0 / (RES - 1))
    inv_wall = 1.0 - wall_ref[...]
    for r in range(NUM_ROOMS):
        cx = rp_ref[r, 0]
        cy = rp_ref[r, 1]
        sx = rp_ref[r, 2]
        sy = rp_ref[r, 3]
        dx = xi - cx
        dy = yj - cy
        e = jnp.exp(-(dx * dx / (2.0 * sx * sx) + dy * dy / (2.0 * sy * sy)))
        out_ref[r] = e * inv_wall


def _tc_layout(room_params, wall_density):
    return pl.pallas_call(
        _layout_body,
        in_specs=[
            pl.BlockSpec(memory_space=pltpu.SMEM),
            pl.BlockSpec((RES, RES), lambda: (0, 0)),
        ],
        out_specs=pl.BlockSpec((NUM_ROOMS, RES, RES), lambda: (0, 0, 0)),
        out_shape=jax.ShapeDtypeStruct((NUM_ROOMS, RES, RES), jnp.float32),
    )(room_params, wall_density)


def _flow_body(p_ref, out_ref):
    h = p_ref[0, :] + p_ref[1, :]
    lin = lax.broadcasted_iota(jnp.int32, (NBINS,), 0)
    h = h - jnp.where(lin == 0, jnp.float32(PAD_COUNT), 0.0)
    m = jnp.max(h)
    out_ref[...] = jnp.reshape(h / (m + 1e-6), (RES, RES))


def _tc_flow(partial):
    return pl.pallas_call(
        _flow_body,
        out_shape=jax.ShapeDtypeStruct((RES, RES), jnp.float32),
    )(partial)


def kernel(agent_positions, room_params, wall_density):
    pos_t = jnp.pad(agent_positions.T, ((0, 0), (0, PAD_N - N_AGENTS)))
    partial = _sc_histogram(pos_t)
    dynamic_layout = _tc_layout(room_params, wall_density)
    flow_field = _tc_flow(partial)
    return dynamic_layout, flow_field


# speedup vs baseline: 5.4698x; 1.0084x over previous
import functools

import jax
import jax.numpy as jnp
from jax import lax
from jax.experimental import pallas as pl
from jax.experimental.pallas import tpu as pltpu
from jax.experimental.pallas import tpu_sc as plsc

RES = 256
NBINS = RES * RES
NUM_ROOMS = 16
N_AGENTS = 200000

NUM_CORES = 2
NUM_SUBCORES = 16
NW = NUM_CORES * NUM_SUBCORES
ROWS = 49
CHUNK = ROWS * 128
PAD_N = NW * CHUNK
PAD_COUNT = PAD_N - N_AGENTS
ZCH = NBINS // NUM_SUBCORES


def _sc_histogram(pos_t):
    mesh = plsc.VectorSubcoreMesh(core_axis_name="c", subcore_axis_name="s")

    @functools.partial(
        pl.kernel,
        out_type=jax.ShapeDtypeStruct((NUM_CORES, NBINS), jnp.float32),
        mesh=mesh,
        scratch_types=[
            pltpu.VMEM((CHUNK,), jnp.float32),
            pltpu.VMEM((CHUNK,), jnp.float32),
            pltpu.VMEM((ROWS, 128), jnp.int32),
            pltpu.VMEM((128,), jnp.float32),
            pltpu.VMEM((ZCH,), jnp.float32),
            pltpu.VMEM_SHARED((NBINS,), jnp.float32),
            pltpu.SemaphoreType.DMA,
            pltpu.SemaphoreType.DMA,
        ],
    )
    def hist_kernel(pos_hbm, out_hbm, x_v, y_v, idx_v, ones_v, zero_v,
                    hist_sh, sem, sem_sc):
        cid = lax.axis_index("c")
        sid = lax.axis_index("s")
        wid = sid * NUM_CORES + cid
        base = wid * CHUNK

        cp_x = pltpu.make_async_copy(pos_hbm.at[0, pl.ds(base, CHUNK)], x_v,
                                     sem)
        cp_y = pltpu.make_async_copy(pos_hbm.at[1, pl.ds(base, CHUNK)], y_v,
                                     sem)
        cp_x.start()
        cp_y.start()

        @pl.loop(0, 128, step=16)
        def _(i):
            ones_v[pl.ds(i, 16)] = jnp.full((16,), 1.0, jnp.float32)

        @pl.loop(0, ZCH, step=16)
        def _(i):
            zero_v[pl.ds(i, 16)] = jnp.zeros((16,), jnp.float32)

        pltpu.sync_copy(zero_v, hist_sh.at[pl.ds(sid * ZCH, ZCH)])
        plsc.subcore_barrier()

        cp_x.wait()
        cp_y.wait()

        @pl.loop(0, ROWS)
        def _(r):
            @pl.loop(0, 128, step=16)
            def _(cc):
                off = r * 128 + cc
                x = x_v[pl.ds(off, 16)]
                y = y_v[pl.ds(off, 16)]
                ix = (x * 256.0).astype(jnp.int32)
                iy = (y * 256.0).astype(jnp.int32)
                ix = jnp.minimum(jnp.maximum(ix, 0), RES - 1)
                iy = jnp.minimum(jnp.maximum(iy, 0), RES - 1)
                idx_v[r, pl.ds(cc, 16)] = ix * RES + iy
            pltpu.async_copy(ones_v, hist_sh.at[idx_v.at[r]], sem_sc,
                             add=True)

        @pl.loop(0, ROWS)
        def _(r):
            pltpu.make_async_copy(ones_v, hist_sh.at[idx_v.at[r]],
                                  sem_sc).wait()

        plsc.subcore_barrier()

        pltpu.sync_copy(hist_sh.at[pl.ds(sid * ZCH, ZCH)],
                        out_hbm.at[cid, pl.ds(sid * ZCH, ZCH)])

    return hist_kernel(pos_t)


def _layout_body(rp_ref, wall_ref, out_ref):
    xi = lax.broadcasted_iota(jnp.int32, (RES, RES), 0).astype(jnp.float32) * (
        1.0 / (RES - 1))
    yj = lax.broadcasted_iota(jnp.int32, (RES, RES), 1).astype(jnp.float32) * (
        1.0 / (RES - 1))
    inv_wall = 1.0 - wall_ref[...]
    for r in range(NUM_ROOMS):
        cx = rp_ref[r, 0]
        cy = rp_ref[r, 1]
        sx = rp_ref[r, 2]
        sy = rp_ref[r, 3]
        dx = xi - cx
        dy = yj - cy
        e = jnp.exp(-(dx * dx / (2.0 * sx * sx) + dy * dy / (2.0 * sy * sy)))
        out_ref[r] = e * inv_wall


def _tc_layout(room_params, wall_density):
    return pl.pallas_call(
        _layout_body,
        in_specs=[
            pl.BlockSpec(memory_space=pltpu.SMEM),
            pl.BlockSpec((RES, RES), lambda: (0, 0)),
        ],
        out_specs=pl.BlockSpec((NUM_ROOMS, RES, RES), lambda: (0, 0, 0)),
        out_shape=jax.ShapeDtypeStruct((NUM_ROOMS, RES, RES), jnp.float32),
    )(room_params, wall_density)


def _flow_body(p_ref, out_ref):
    h = p_ref[0, :] + p_ref[1, :]
    lin = lax.broadcasted_iota(jnp.int32, (NBINS,), 0)
    h = h - jnp.where(lin == 0, jnp.float32(PAD_COUNT), 0.0)
    m = jnp.max(h)
    out_ref[...] = jnp.reshape(h / (m + 1e-6), (RES, RES))


def _tc_flow(partial):
    return pl.pallas_call(
        _flow_body,
        out_shape=jax.ShapeDtypeStruct((RES, RES), jnp.float32),
    )(partial)


def kernel(agent_positions, room_params, wall_density):
    pos_t = jnp.pad(agent_positions.T, ((0, 0), (0, PAD_N - N_AGENTS)))
    partial = _sc_histogram(pos_t)
    dynamic_layout = _tc_layout(room_params, wall_density)
    flow_field = _tc_flow(partial)
    return dynamic_layout, flow_field
